# baseline (speedup 1.0000x reference)
import jax
import jax.numpy as jnp
from jax import lax
from jax.experimental import pallas as pl
from jax.experimental.pallas import tpu as pltpu
from jax.experimental.pallas import tpu_sc as plsc

EMBED = 32
NC = 2
NS = 16
NW = NC * NS
BT = 128
HIST = 50


def _build(B, V):
    n_bt = B // HIST // BT
    bt_per_w = n_bt // NW
    n_items = bt_per_w * HIST

    def body(table_hbm, idxT_hbm, x_hbm, idx_v, rows_v, tb_v,
             sg0, sg1, ss0, ss1):
        wid = lax.axis_index("s") * NC + lax.axis_index("c")
        bt0 = wid * bt_per_w

        pltpu.sync_copy(idxT_hbm.at[:, pl.ds(bt0 * BT, bt_per_w * BT)],
                        idx_v)

        def coords(k):
            return k // HIST, lax.rem(k, HIST)

        def gather_start(k, buf, sem):
            btl, h = coords(k)
            pltpu.async_copy(
                table_hbm.at[idx_v.at[h, pl.ds(btl * BT, BT)]],
                rows_v.at[buf], sem)

        def gather_wait(k, buf, sem):
            btl, h = coords(k)
            pltpu.make_async_copy(
                table_hbm.at[idx_v.at[h, pl.ds(btl * BT, BT)]],
                rows_v.at[buf], sem).wait()

        def transpose(buf):
            for e in range(EMBED):
                col = jnp.full((16,), e, jnp.int32)
                for g in range(8):
                    rowsidx = lax.iota(jnp.int32, 16) + (16 * g)
                    v = plsc.load_gather(rows_v.at[buf], [rowsidx, col])
                    tb_v[buf, e // 8, e % 8, pl.ds(g * 16, 16)] = v

        def out_start(k, buf, sem):
            btl, h = coords(k)
            for et in range(4):
                pltpu.async_copy(tb_v.at[buf, et],
                                 x_hbm.at[h, et, bt0 + btl], sem)

        def out_wait(k, buf, sem):
            btl, h = coords(k)
            for et in range(4):
                pltpu.make_async_copy(tb_v.at[buf, et],
                                      x_hbm.at[h, et, bt0 + btl],
                                      sem).wait()

        gather_start(0, 0, sg0)
        gather_start(1, 1, sg1)

        def item(k, buf, sg, ss, kk):
            gather_wait(k, buf, sg)

            @pl.when(kk > 0)
            def _():
                out_wait(k - 2, buf, ss)
            transpose(buf)
            out_start(k, buf, ss)

            @pl.when(kk < n_items // 2 - 1)
            def _():
                gather_start(k + 2, buf, sg)

        def outer(kk, carry):
            k0 = 2 * kk
            item(k0, 0, sg0, ss0, kk)
            item(k0 + 1, 1, sg1, ss1, kk)
            return carry

        lax.fori_loop(0, n_items // 2, outer, 0)
        out_wait(n_items - 2, 0, ss0)
        out_wait(n_items - 1, 1, ss1)

    mesh = plsc.VectorSubcoreMesh(
        core_axis_name="c", subcore_axis_name="s", num_cores=NC,
        num_subcores=NS,
    )
    return pl.kernel(
        body,
        out_type=jax.ShapeDtypeStruct(
            (HIST, EMBED // 8, n_bt, 8, BT), jnp.float32),
        mesh=mesh,
        compiler_params=pltpu.CompilerParams(
            use_tc_tiling_on_sc=False, needs_layout_passes=False
        ),
        scratch_types=[
            pltpu.VMEM((HIST, bt_per_w * BT), jnp.int32),
            pltpu.VMEM((2, BT, EMBED), jnp.float32),
            pltpu.VMEM((2, EMBED // 8, 8, BT), jnp.float32),
            pltpu.SemaphoreType.DMA,
            pltpu.SemaphoreType.DMA,
            pltpu.SemaphoreType.DMA,
            pltpu.SemaphoreType.DMA,
        ],
    )


def kernel(inputs, table):
    B = inputs.size
    idxT = inputs.T.astype(jnp.int32)
    x = _build(B, table.shape[0])(table, idxT)
    out = x.transpose(2, 4, 0, 1, 3).reshape(B // HIST, HIST, EMBED)
    return out


# speedup vs baseline: 1.1964x; 1.1964x over previous
import jax
import jax.numpy as jnp
from jax import lax
from jax.experimental import pallas as pl
from jax.experimental.pallas import tpu as pltpu
from jax.experimental.pallas import tpu_sc as plsc

EMBED = 32
NC = 2
NS = 16
NW = NC * NS
BT = 128
HIST = 50


def _build(B, V):
    n_bt = B // HIST // BT
    bt_per_w = n_bt // NW
    n_items = bt_per_w * HIST

    def body(table_hbm, idxT_hbm, x_hbm, idx_v, rows_v, tb_v,
             sg0, sg1, sg2, sg3, ss0, ss1, ss2, ss3):
        wid = lax.axis_index("s") * NC + lax.axis_index("c")
        bt0 = wid * bt_per_w

        pltpu.sync_copy(idxT_hbm.at[:, pl.ds(bt0 * BT, bt_per_w * BT)],
                        idx_v)

        def coords(k):
            return k // HIST, lax.rem(k, HIST)

        def gather_start(k, buf, sem):
            btl, h = coords(k)
            pltpu.async_copy(
                table_hbm.at[idx_v.at[h, pl.ds(btl * BT, BT)]],
                rows_v.at[buf], sem)

        def gather_wait(k, buf, sem):
            btl, h = coords(k)
            pltpu.make_async_copy(
                table_hbm.at[idx_v.at[h, pl.ds(btl * BT, BT)]],
                rows_v.at[buf], sem).wait()

        def transpose(buf, tbuf):
            for e in range(EMBED):
                col = jnp.full((16,), e, jnp.int32)
                vs = []
                for g in range(8):
                    rowsidx = lax.iota(jnp.int32, 16) + (16 * g)
                    vs.append(
                        plsc.load_gather(rows_v.at[buf], [rowsidx, col]))
                for g in range(8):
                    tb_v[tbuf, e // 8, e % 8, pl.ds(g * 16, 16)] = vs[g]

        def out_start(k, tbuf, sem):
            btl, h = coords(k)
            for et in range(4):
                pltpu.async_copy(tb_v.at[tbuf, et],
                                 x_hbm.at[h, et, bt0 + btl], sem)

        def out_wait(k, tbuf, sem):
            btl, h = coords(k)
            for et in range(4):
                pltpu.make_async_copy(tb_v.at[tbuf, et],
                                      x_hbm.at[h, et, bt0 + btl],
                                      sem).wait()

        sgs = (sg0, sg1, sg2, sg3)
        sss = (ss0, ss1, ss2, ss3)
        for j in range(4):
            gather_start(j, j, sgs[j])

        def item(k, buf, kk):
            gather_wait(k, buf, sgs[buf])

            @pl.when(kk > 0)
            def _():
                out_wait(k - 4, buf, sss[buf])
            transpose(buf, buf)
            out_start(k, buf, sss[buf])

            @pl.when(kk < n_items // 4 - 1)
            def _():
                gather_start(k + 4, buf, sgs[buf])

        def outer(kk, carry):
            k0 = 4 * kk
            for j in range(4):
                item(k0 + j, j, kk)
            return carry

        lax.fori_loop(0, n_items // 4, outer, 0)
        for j in range(4):
            out_wait(n_items - 4 + j, j, sss[j])

    mesh = plsc.VectorSubcoreMesh(
        core_axis_name="c", subcore_axis_name="s", num_cores=NC,
        num_subcores=NS,
    )
    return pl.kernel(
        body,
        out_type=jax.ShapeDtypeStruct(
            (HIST, EMBED // 8, n_bt, 8, BT), jnp.float32),
        mesh=mesh,
        compiler_params=pltpu.CompilerParams(
            use_tc_tiling_on_sc=False, needs_layout_passes=False
        ),
        scratch_types=[
            pltpu.VMEM((HIST, bt_per_w * BT), jnp.int32),
            pltpu.VMEM((4, BT, EMBED), jnp.float32),
            pltpu.VMEM((4, EMBED // 8, 8, BT), jnp.float32),
            pltpu.SemaphoreType.DMA,
            pltpu.SemaphoreType.DMA,
            pltpu.SemaphoreType.DMA,
            pltpu.SemaphoreType.DMA,
            pltpu.SemaphoreType.DMA,
            pltpu.SemaphoreType.DMA,
            pltpu.SemaphoreType.DMA,
            pltpu.SemaphoreType.DMA,
        ],
    )


def kernel(inputs, table):
    B = inputs.size
    idxT = inputs.T.astype(jnp.int32)
    x = _build(B, table.shape[0])(table, idxT)
    out = x.transpose(2, 4, 0, 1, 3).reshape(B // HIST, HIST, EMBED)
    return out


# speedup vs baseline: 1.1996x; 1.0027x over previous
import jax
import jax.numpy as jnp
from jax import lax
from jax.experimental import pallas as pl
from jax.experimental.pallas import tpu as pltpu
from jax.experimental.pallas import tpu_sc as plsc

EMBED = 32
NC = 2
NS = 16
NW = NC * NS
BT = 128
HIST = 50


def _build(B, V):
    n_bt = B // HIST // BT
    bt_per_w = n_bt // NW
    bw = bt_per_w * BT

    def body(table_hbm, idxT_hbm, x_hbm, idx_v, rows_v, tb_v,
             sg0, sg1, ss0, ss1):
        wid = lax.axis_index("s") * NC + lax.axis_index("c")
        bt0 = wid * bt_per_w

        pltpu.sync_copy(idxT_hbm.at[:, pl.ds(bt0 * BT, bw)], idx_v)

        def gather_start(h, buf, sem):
            pltpu.async_copy(table_hbm.at[idx_v.at[h]], rows_v.at[buf],
                             sem)

        def gather_wait(h, buf, sem):
            pltpu.make_async_copy(table_hbm.at[idx_v.at[h]],
                                  rows_v.at[buf], sem).wait()

        def transpose(buf, tbuf):
            for btl in range(bt_per_w):
                for e in range(EMBED):
                    col = jnp.full((16,), e, jnp.int32)
                    vs = []
                    for g in range(8):
                        rowsidx = lax.iota(jnp.int32, 16) + (
                            btl * BT + 16 * g)
                        vs.append(plsc.load_gather(
                            rows_v.at[buf], [rowsidx, col]))
                    for g in range(8):
                        tb_v[tbuf, e // 8, btl, e % 8,
                             pl.ds(g * 16, 16)] = vs[g]

        def out_start(h, tbuf, sem):
            pltpu.async_copy(tb_v.at[tbuf],
                             x_hbm.at[h, :, pl.ds(bt0, bt_per_w)], sem)

        def out_wait(h, tbuf, sem):
            pltpu.make_async_copy(tb_v.at[tbuf],
                                  x_hbm.at[h, :, pl.ds(bt0, bt_per_w)],
                                  sem).wait()

        sgs = (sg0, sg1)
        sss = (ss0, ss1)
        gather_start(0, 0, sg0)
        gather_start(1, 1, sg1)

        def step(h, buf, kk):
            gather_wait(h, buf, sgs[buf])

            @pl.when(kk > 0)
            def _():
                out_wait(h - 2, buf, sss[buf])
            transpose(buf, buf)
            out_start(h, buf, sss[buf])

            @pl.when(h + 2 < HIST)
            def _():
                gather_start(h + 2, buf, sgs[buf])

        def outer(kk, carry):
            step(2 * kk, 0, kk)
            step(2 * kk + 1, 1, kk)
            return carry

        lax.fori_loop(0, HIST // 2, outer, 0)
        out_wait(HIST - 2, 0, ss0)
        out_wait(HIST - 1, 1, ss1)

    mesh = plsc.VectorSubcoreMesh(
        core_axis_name="c", subcore_axis_name="s", num_cores=NC,
        num_subcores=NS,
    )
    return pl.kernel(
        body,
        out_type=jax.ShapeDtypeStruct(
            (HIST, EMBED // 8, n_bt, 8, BT), jnp.float32),
        mesh=mesh,
        compiler_params=pltpu.CompilerParams(
            use_tc_tiling_on_sc=False, needs_layout_passes=False
        ),
        scratch_types=[
            pltpu.VMEM((HIST, bw), jnp.int32),
            pltpu.VMEM((2, bw, EMBED), jnp.float32),
            pltpu.VMEM((2, EMBED // 8, bt_per_w, 8, BT), jnp.float32),
            pltpu.SemaphoreType.DMA,
            pltpu.SemaphoreType.DMA,
            pltpu.SemaphoreType.DMA,
            pltpu.SemaphoreType.DMA,
        ],
    )


def kernel(inputs, table):
    B = inputs.size
    idxT = inputs.T.astype(jnp.int32)
    x = _build(B, table.shape[0])(table, idxT)
    out = x.transpose(2, 4, 0, 1, 3).reshape(B // HIST, HIST, EMBED)
    return out


# speedup vs baseline: 1.7092x; 1.4248x over previous
import jax
import jax.numpy as jnp
from jax import lax
from jax.experimental import pallas as pl
from jax.experimental.pallas import tpu as pltpu
from jax.experimental.pallas import tpu_sc as plsc

EMBED = 32
NC = 2
NS = 16
NW = NC * NS
BT = 128
HIST = 50


def _build(B, V):
    n_bt = B // HIST // BT
    bt_per_w = n_bt // NW
    bw = bt_per_w * BT

    def body(table_hbm, idxT_hbm, x_hbm, idx_v, rows_v, tb_v,
             sg0, sg1, ss0, ss1):
        wid = lax.axis_index("s") * NC + lax.axis_index("c")
        bt0 = wid * bt_per_w

        pltpu.sync_copy(idxT_hbm.at[:, pl.ds(bt0 * BT, bw)], idx_v)

        def gather_start(h, buf, sem):
            pltpu.async_copy(table_hbm.at[idx_v.at[h]], rows_v.at[buf],
                             sem)

        def gather_wait(h, buf, sem):
            pltpu.make_async_copy(table_hbm.at[idx_v.at[h]],
                                  rows_v.at[buf], sem).wait()

        e0 = lax.iota(jnp.int32, 16)

        def transpose(buf, tbuf):
            def l_body(l0, carry):
                for j in range(16):
                    l = l0 * 16 + j
                    btl = l // BT
                    bl = lax.rem(l, BT)
                    bv = jnp.zeros((16,), jnp.int32) + btl
                    lv = jnp.zeros((16,), jnp.int32) + bl
                    for g2 in range(2):
                        ev = e0 + 16 * g2
                        v = rows_v[buf, l, pl.ds(g2 * 16, 16)]
                        plsc.store_scatter(
                            tb_v.at[tbuf],
                            [ev // 8, bv, lax.rem(ev, 8), lv], v)
                return carry
            lax.fori_loop(0, bw // 16, l_body, 0)

        def out_start(h, tbuf, sem):
            pltpu.async_copy(tb_v.at[tbuf, :, :, :, pl.ds(0, BT)],
                             x_hbm.at[h, :, pl.ds(bt0, bt_per_w)], sem)

        def out_wait(h, tbuf, sem):
            pltpu.make_async_copy(tb_v.at[tbuf, :, :, :, pl.ds(0, BT)],
                                  x_hbm.at[h, :, pl.ds(bt0, bt_per_w)],
                                  sem).wait()

        sgs = (sg0, sg1)
        sss = (ss0, ss1)
        gather_start(0, 0, sg0)
        gather_start(1, 1, sg1)

        def step(h, buf, kk):
            gather_wait(h, buf, sgs[buf])

            @pl.when(kk > 0)
            def _():
                out_wait(h - 2, buf, sss[buf])
            transpose(buf, buf)
            out_start(h, buf, sss[buf])

            @pl.when(h + 2 < HIST)
            def _():
                gather_start(h + 2, buf, sgs[buf])

        def outer(kk, carry):
            step(2 * kk, 0, kk)
            step(2 * kk + 1, 1, kk)
            return carry

        lax.fori_loop(0, HIST // 2, outer, 0)
        out_wait(HIST - 2, 0, ss0)
        out_wait(HIST - 1, 1, ss1)

    mesh = plsc.VectorSubcoreMesh(
        core_axis_name="c", subcore_axis_name="s", num_cores=NC,
        num_subcores=NS,
    )
    return pl.kernel(
        body,
        out_type=jax.ShapeDtypeStruct(
            (HIST, EMBED // 8, n_bt, 8, BT), jnp.float32),
        mesh=mesh,
        compiler_params=pltpu.CompilerParams(
            use_tc_tiling_on_sc=False, needs_layout_passes=False
        ),
        scratch_types=[
            pltpu.VMEM((HIST, bw), jnp.int32),
            pltpu.VMEM((2, bw, EMBED), jnp.float32),
            pltpu.VMEM((2, EMBED // 8, bt_per_w, 8, BT + 1), jnp.float32),
            pltpu.SemaphoreType.DMA,
            pltpu.SemaphoreType.DMA,
            pltpu.SemaphoreType.DMA,
            pltpu.SemaphoreType.DMA,
        ],
    )


def kernel(inputs, table):
    B = inputs.size
    idxT = inputs.T.astype(jnp.int32)
    x = _build(B, table.shape[0])(table, idxT)
    out = x.transpose(2, 4, 0, 1, 3).reshape(B // HIST, HIST, EMBED)
    return out


# speedup vs baseline: 1.7128x; 1.0021x over previous
import jax
import jax.numpy as jnp
from jax import lax
from jax.experimental import pallas as pl
from jax.experimental.pallas import tpu as pltpu
from jax.experimental.pallas import tpu_sc as plsc

EMBED = 32
PADW = 128
NC = 2
NS = 16
NW = NC * NS
BT = 128
HIST = 50
SUB = 256


def _build(B, V):
    n_bt = B // HIST // BT
    bt_per_w = n_bt // NW
    bw = bt_per_w * BT

    def body(table_hbm, idxT_hbm, x_hbm, idx_v, rows_v, tb_v,
             sg0, sg1, ss0, ss1):
        wid = lax.axis_index("s") * NC + lax.axis_index("c")
        bt0 = wid * bt_per_w

        pltpu.sync_copy(idxT_hbm.at[:, pl.ds(bt0 * BT, bw)], idx_v)

        def gather_start(h, p, buf, sem):
            pltpu.async_copy(
                table_hbm.at[idx_v.at[h, pl.ds(p * SUB, SUB)]],
                rows_v.at[buf], sem)

        def gather_wait(h, p, buf, sem):
            pltpu.make_async_copy(
                table_hbm.at[idx_v.at[h, pl.ds(p * SUB, SUB)]],
                rows_v.at[buf], sem).wait()

        e0 = lax.iota(jnp.int32, 16)

        def transpose(buf, tbuf):
            def l_body(l0, carry):
                for j in range(16):
                    l = l0 * 16 + j
                    bv = jnp.zeros((16,), jnp.int32) + l // BT
                    lv = jnp.zeros((16,), jnp.int32) + lax.rem(l, BT)
                    for g2 in range(2):
                        ev = e0 + 16 * g2
                        v = rows_v[buf, l, pl.ds(g2 * 16, 16)]
                        plsc.store_scatter(
                            tb_v.at[tbuf],
                            [ev // 8, bv, lax.rem(ev, 8), lv], v)
                return carry
            lax.fori_loop(0, SUB // 16, l_body, 0)

        def out_start(h, p, tbuf, sem):
            pltpu.async_copy(
                tb_v.at[tbuf, :, :, :, pl.ds(0, BT)],
                x_hbm.at[h, :, pl.ds(bt0 + 2 * p, 2)], sem)

        def out_wait(h, p, tbuf, sem):
            pltpu.make_async_copy(
                tb_v.at[tbuf, :, :, :, pl.ds(0, BT)],
                x_hbm.at[h, :, pl.ds(bt0 + 2 * p, 2)], sem).wait()

        sgs = (sg0, sg1)
        sss = (ss0, ss1)
        gather_start(0, 0, 0, sg0)
        gather_start(0, 1, 1, sg1)

        def step(h, p, buf, kk):
            gather_wait(h, p, buf, sgs[buf])

            @pl.when(kk > 0)
            def _():
                out_wait(h - 1, p, buf, sss[buf])
            transpose(buf, buf)
            out_start(h, p, buf, sss[buf])

            @pl.when(h + 1 < HIST)
            def _():
                gather_start(h + 1, p, buf, sgs[buf])

        def outer(kk, carry):
            step(kk, 0, 0, kk)
            step(kk, 1, 1, kk)
            return carry

        lax.fori_loop(0, HIST, outer, 0)
        out_wait(HIST - 1, 0, 0, ss0)
        out_wait(HIST - 1, 1, 1, ss1)

    mesh = plsc.VectorSubcoreMesh(
        core_axis_name="c", subcore_axis_name="s", num_cores=NC,
        num_subcores=NS,
    )
    return pl.kernel(
        body,
        out_type=jax.ShapeDtypeStruct(
            (HIST, EMBED // 8, n_bt, 8, BT), jnp.float32),
        mesh=mesh,
        compiler_params=pltpu.CompilerParams(
            use_tc_tiling_on_sc=False, needs_layout_passes=False
        ),
        scratch_types=[
            pltpu.VMEM((HIST, bw), jnp.int32),
            pltpu.VMEM((2, SUB, PADW), jnp.float32),
            pltpu.VMEM((2, EMBED // 8, 2, 8, BT + 1), jnp.float32),
            pltpu.SemaphoreType.DMA,
            pltpu.SemaphoreType.DMA,
            pltpu.SemaphoreType.DMA,
            pltpu.SemaphoreType.DMA,
        ],
    )


def kernel(inputs, table):
    B = inputs.size
    V = table.shape[0]
    idxT = inputs.T.astype(jnp.int32)
    tpad = jnp.pad(
        table.reshape(V // 8, 8, EMBED), ((0, 0), (0, 0), (0, PADW - EMBED))
    ).reshape(V, PADW)
    x = _build(B, V)(tpad, idxT)
    out = x.transpose(2, 4, 0, 1, 3).reshape(B // HIST, HIST, EMBED)
    return out
